# Initial kernel scaffold; baseline (speedup 1.0000x reference)
import functools

import jax
import jax.numpy as jnp
from jax import lax
from jax.experimental import pallas as pl
from jax.experimental.pallas import tpu as pltpu
from jax.experimental.pallas import tpu_sc as plsc

N = 10000
E = 320000
D = 128
H = 128
NC, NS = 2, 16
NW = NC * NS
FPT = D // NW
EPT = E // NW
L = 16
CH = 4000
RB = 1000

_mesh = plsc.VectorSubcoreMesh(
    core_axis_name="c", subcore_axis_name="s", num_cores=NC, num_subcores=NS)


def _worker_id():
    return lax.axis_index("s") * NC + lax.axis_index("c")


@functools.partial(
    pl.kernel,
    out_type=jax.ShapeDtypeStruct((NW, N), jnp.float32),
    mesh=_mesh,
    scratch_types=[
        pltpu.VMEM((EPT,), jnp.int32),
        pltpu.VMEM((EPT,), jnp.float32),
        pltpu.VMEM((N,), jnp.float32),
    ],
)
def _sc_deg(dst_hbm, ew_hbm, out_hbm, dstv, eww, acc):
    w = _worker_id()
    base = w * EPT
    pltpu.sync_copy(dst_hbm.at[pl.ds(base, EPT)], dstv)
    pltpu.sync_copy(ew_hbm.at[pl.ds(base, EPT)], eww)
    zero = jnp.zeros((L,), jnp.float32)

    def zbody(i, carry):
        acc[pl.ds(i * L, L)] = zero
        return carry
    lax.fori_loop(0, N // L, zbody, 0)

    def body(i, carry):
        d16 = dstv[pl.ds(i * L, L)]
        w16 = eww[pl.ds(i * L, L)]
        plsc.addupdate_scatter(acc, [d16], w16)
        return carry
    lax.fori_loop(0, EPT // L, body, 0)
    pltpu.sync_copy(acc, out_hbm.at[w])


@functools.partial(
    pl.kernel,
    out_type=jax.ShapeDtypeStruct((NW, N, FPT), jnp.float32),
    mesh=_mesh,
    scratch_types=[
        pltpu.VMEM((N, FPT), jnp.float32),
        pltpu.VMEM((N, FPT), jnp.float32),
        pltpu.VMEM((CH,), jnp.int32),
        pltpu.VMEM((CH,), jnp.int32),
        pltpu.VMEM((CH,), jnp.float32),
    ],
)
def _sc_agg(xsc_hbm, src_hbm, dst_hbm, ew_hbm, out_hbm,
            table, acc, srcb, dstb, ewb):
    w = _worker_id()
    pltpu.sync_copy(xsc_hbm.at[w], table)
    zero = jnp.zeros((L,), jnp.float32)
    niota = lax.iota(jnp.int32, L)
    f16s = [jnp.full((L,), f, jnp.int32) for f in range(FPT)]

    def zbody(i, carry):
        n16 = niota + i * L
        for f in range(FPT):
            plsc.store_scatter(acc, [n16, f16s[f]], zero)
        return carry
    lax.fori_loop(0, N // L, zbody, 0)

    def obody(ci, carry):
        off = ci * CH
        pltpu.sync_copy(src_hbm.at[pl.ds(off, CH)], srcb)
        pltpu.sync_copy(dst_hbm.at[pl.ds(off, CH)], dstb)
        pltpu.sync_copy(ew_hbm.at[pl.ds(off, CH)], ewb)

        def ibody(j, icarry):
            s16 = srcb[pl.ds(j * L, L)]
            d16 = dstb[pl.ds(j * L, L)]
            w16 = ewb[pl.ds(j * L, L)]
            for f in range(FPT):
                v = plsc.load_gather(table, [s16, f16s[f]])
                plsc.addupdate_scatter(acc, [d16, f16s[f]], v * w16)
            return icarry
        lax.fori_loop(0, CH // L, ibody, carry)
        return carry
    lax.fori_loop(0, E // CH, obody, 0)
    pltpu.sync_copy(acc, out_hbm.at[w])


@functools.partial(
    pl.kernel,
    out_type=jax.ShapeDtypeStruct((NW, N), jnp.float32),
    mesh=_mesh,
    scratch_types=[
        pltpu.VMEM((N,), jnp.float32),
        pltpu.VMEM((EPT,), jnp.int32),
        pltpu.VMEM((EPT,), jnp.int32),
        pltpu.VMEM((EPT,), jnp.float32),
        pltpu.VMEM((N,), jnp.float32),
    ],
)
def _sc_agg2(zs_hbm, src_hbm, dst_hbm, ew_hbm, out_hbm,
             zsv, srcv, dstv, eww, acc):
    w = _worker_id()
    base = w * EPT
    pltpu.sync_copy(zs_hbm, zsv)
    pltpu.sync_copy(src_hbm.at[pl.ds(base, EPT)], srcv)
    pltpu.sync_copy(dst_hbm.at[pl.ds(base, EPT)], dstv)
    pltpu.sync_copy(ew_hbm.at[pl.ds(base, EPT)], eww)
    zero = jnp.zeros((L,), jnp.float32)

    def zbody(i, carry):
        acc[pl.ds(i * L, L)] = zero
        return carry
    lax.fori_loop(0, N // L, zbody, 0)

    def body(i, carry):
        s16 = srcv[pl.ds(i * L, L)]
        d16 = dstv[pl.ds(i * L, L)]
        w16 = eww[pl.ds(i * L, L)]
        v = plsc.load_gather(zsv, [s16])
        plsc.addupdate_scatter(acc, [d16], v * w16)
        return carry
    lax.fori_loop(0, EPT // L, body, 0)
    pltpu.sync_copy(acc, out_hbm.at[w])


def _tc1_body(deg_ref, x_ref, w1_ref, xs_ref, dinv_ref):
    j = pl.program_id(0)
    deg = jnp.sum(deg_ref[...], axis=0) + 1.0
    degc = lax.dynamic_slice(deg, (j * RB,), (RB,))
    dinv = lax.rsqrt(degc)
    xw = jnp.dot(x_ref[...], w1_ref[...], preferred_element_type=jnp.float32)
    xs_ref[...] = xw * dinv[:, None]
    dinv_ref[...] = dinv[:, None]


def _tc1(deg_part, x, W1):
    return pl.pallas_call(
        _tc1_body,
        grid=(N // RB,),
        in_specs=[
            pl.BlockSpec((NW, N), lambda j: (0, 0)),
            pl.BlockSpec((RB, D), lambda j: (j, 0)),
            pl.BlockSpec((D, H), lambda j: (0, 0)),
        ],
        out_specs=[
            pl.BlockSpec((RB, H), lambda j: (j, 0)),
            pl.BlockSpec((RB, 1), lambda j: (j, 0)),
        ],
        out_shape=[
            jax.ShapeDtypeStruct((N, H), jnp.float32),
            jax.ShapeDtypeStruct((N, 1), jnp.float32),
        ],
    )(deg_part, x, W1)


def _tc2_body(agg_ref, xs_ref, dinv_ref, b1_ref, w2_ref, zs_ref):
    dinv = dinv_ref[...]
    h = jnp.maximum(dinv * (agg_ref[...] + xs_ref[...]) + b1_ref[...], 0.0)
    z = jnp.dot(h, w2_ref[...], preferred_element_type=jnp.float32)
    zs_ref[...] = dinv * z


def _tc2(agg, xs, dinv, b1, W2):
    return pl.pallas_call(
        _tc2_body,
        grid=(N // RB,),
        in_specs=[
            pl.BlockSpec((RB, H), lambda j: (j, 0)),
            pl.BlockSpec((RB, H), lambda j: (j, 0)),
            pl.BlockSpec((RB, 1), lambda j: (j, 0)),
            pl.BlockSpec((1, H), lambda j: (0, 0)),
            pl.BlockSpec((H, 1), lambda j: (0, 0)),
        ],
        out_specs=pl.BlockSpec((RB, 1), lambda j: (j, 0)),
        out_shape=jax.ShapeDtypeStruct((N, 1), jnp.float32),
    )(agg, xs, dinv, b1, W2)


def _tc3_body(a2_ref, zs_ref, dinv_ref, b2_ref, out_ref):
    j = pl.program_id(0)
    a2 = jnp.sum(a2_ref[...], axis=0)
    a2c = lax.dynamic_slice(a2, (j * RB,), (RB,))
    out_ref[...] = dinv_ref[...] * (a2c[:, None] + zs_ref[...]) + b2_ref[...]


def _tc3(agg2_part, zs, dinv, b2):
    return pl.pallas_call(
        _tc3_body,
        grid=(N // RB,),
        in_specs=[
            pl.BlockSpec((NW, N), lambda j: (0, 0)),
            pl.BlockSpec((RB, 1), lambda j: (j, 0)),
            pl.BlockSpec((RB, 1), lambda j: (j, 0)),
            pl.BlockSpec((1, 1), lambda j: (0, 0)),
        ],
        out_specs=pl.BlockSpec((RB, 1), lambda j: (j, 0)),
        out_shape=jax.ShapeDtypeStruct((N, 1), jnp.float32),
    )(agg2_part, zs, dinv, b2)


def kernel(x, edge_index, edge_weight, W1, b1, W2, b2):
    src = edge_index[0].astype(jnp.int32)
    dst = edge_index[1].astype(jnp.int32)
    ew = edge_weight.astype(jnp.float32)

    deg_part = _sc_deg(dst, ew)
    xs, dinv = _tc1(deg_part, x, W1)
    xs_sc = xs.reshape(N, NW, FPT).transpose(1, 0, 2)
    agg_sc = _sc_agg(xs_sc, src, dst, ew)
    agg = agg_sc.transpose(1, 0, 2).reshape(N, D)
    zs = _tc2(agg, xs, dinv, b1.reshape(1, H), W2)
    agg2_part = _sc_agg2(zs.reshape(N), src, dst, ew)
    return _tc3(agg2_part, zs, dinv, b2.reshape(1, 1))


# speedup vs baseline: 9.0645x; 9.0645x over previous
import functools

import jax
import jax.numpy as jnp
from jax import lax
from jax.experimental import pallas as pl
from jax.experimental.pallas import tpu as pltpu
from jax.experimental.pallas import tpu_sc as plsc

N = 10000
E = 320000
D = 128
H = 128
NC, NS = 2, 16
NW = NC * NS
FPT = D // NW
EPT = E // NW
L = 16
CH = 4000
RB = 1000

def _worker_id():
    return lax.axis_index("s") * NC + lax.axis_index("c")


def _mesh():
    return plsc.VectorSubcoreMesh(
        core_axis_name="c", subcore_axis_name="s",
        num_cores=NC, num_subcores=NS)


def _sc_deg_body(dst_hbm, ew_hbm, out_hbm, dstv, eww, acc):
    w = _worker_id()
    base = w * EPT
    pltpu.sync_copy(dst_hbm.at[pl.ds(base, EPT)], dstv)
    pltpu.sync_copy(ew_hbm.at[pl.ds(base, EPT)], eww)
    zero = jnp.zeros((L,), jnp.float32)

    def zbody(i, carry):
        acc[pl.ds(i * L, L)] = zero
        return carry
    lax.fori_loop(0, N // L, zbody, 0)

    def body(i, carry):
        d16 = dstv[pl.ds(i * L, L)]
        w16 = eww[pl.ds(i * L, L)]
        plsc.addupdate_scatter(acc, [d16], w16)
        return carry
    lax.fori_loop(0, EPT // L, body, 0)
    pltpu.sync_copy(acc, out_hbm.at[w])


@functools.cache
def _sc_deg():
    return pl.kernel(
        _sc_deg_body,
        out_type=jax.ShapeDtypeStruct((NW, N), jnp.float32),
        mesh=_mesh(),
        compiler_params=pltpu.CompilerParams(needs_layout_passes=False),
        scratch_types=[
            pltpu.VMEM((EPT,), jnp.int32),
            pltpu.VMEM((EPT,), jnp.float32),
            pltpu.VMEM((N,), jnp.float32),
        ],
    )


def _sc_agg_body(xsc_hbm, src_hbm, dst_hbm, ew_hbm, out_hbm,
                 table, acc, srcb, dstb, ewb):
    w = _worker_id()
    pltpu.sync_copy(xsc_hbm.at[w], table)
    zero = jnp.zeros((L,), jnp.float32)

    def zbody(i, carry):
        acc[pl.ds(i * L, L)] = zero
        return carry
    lax.fori_loop(0, (N * FPT) // L, zbody, 0)

    def obody(ci, carry):
        off = ci * CH
        pltpu.sync_copy(src_hbm.at[pl.ds(off, CH)], srcb)
        pltpu.sync_copy(dst_hbm.at[pl.ds(off, CH)], dstb)
        pltpu.sync_copy(ew_hbm.at[pl.ds(off, CH)], ewb)

        def ibody(j, icarry):
            s16 = srcb[pl.ds(j * L, L)] * FPT
            d16 = dstb[pl.ds(j * L, L)] * FPT
            w16 = ewb[pl.ds(j * L, L)]
            for f in range(FPT):
                v = plsc.load_gather(table, [s16 + f])
                plsc.addupdate_scatter(acc, [d16 + f], v * w16)
            return icarry
        lax.fori_loop(0, CH // L, ibody, carry)
        return carry
    lax.fori_loop(0, E // CH, obody, 0)
    pltpu.sync_copy(acc, out_hbm.at[w])


@functools.cache
def _sc_agg():
    return pl.kernel(
        _sc_agg_body,
        out_type=jax.ShapeDtypeStruct((NW, N * FPT), jnp.float32),
        mesh=_mesh(),
        compiler_params=pltpu.CompilerParams(needs_layout_passes=False),
        scratch_types=[
            pltpu.VMEM((N * FPT,), jnp.float32),
            pltpu.VMEM((N * FPT,), jnp.float32),
            pltpu.VMEM((CH,), jnp.int32),
            pltpu.VMEM((CH,), jnp.int32),
            pltpu.VMEM((CH,), jnp.float32),
        ],
    )


def _sc_agg2_body(zs_hbm, src_hbm, dst_hbm, ew_hbm, out_hbm,
                  zsv, srcv, dstv, eww, acc):
    w = _worker_id()
    base = w * EPT
    pltpu.sync_copy(zs_hbm, zsv)
    pltpu.sync_copy(src_hbm.at[pl.ds(base, EPT)], srcv)
    pltpu.sync_copy(dst_hbm.at[pl.ds(base, EPT)], dstv)
    pltpu.sync_copy(ew_hbm.at[pl.ds(base, EPT)], eww)
    zero = jnp.zeros((L,), jnp.float32)

    def zbody(i, carry):
        acc[pl.ds(i * L, L)] = zero
        return carry
    lax.fori_loop(0, N // L, zbody, 0)

    def body(i, carry):
        s16 = srcv[pl.ds(i * L, L)]
        d16 = dstv[pl.ds(i * L, L)]
        w16 = eww[pl.ds(i * L, L)]
        v = plsc.load_gather(zsv, [s16])
        plsc.addupdate_scatter(acc, [d16], v * w16)
        return carry
    lax.fori_loop(0, EPT // L, body, 0)
    pltpu.sync_copy(acc, out_hbm.at[w])


@functools.cache
def _sc_agg2():
    return pl.kernel(
        _sc_agg2_body,
        out_type=jax.ShapeDtypeStruct((NW, N), jnp.float32),
        mesh=_mesh(),
        compiler_params=pltpu.CompilerParams(needs_layout_passes=False),
        scratch_types=[
            pltpu.VMEM((N,), jnp.float32),
            pltpu.VMEM((EPT,), jnp.int32),
            pltpu.VMEM((EPT,), jnp.int32),
            pltpu.VMEM((EPT,), jnp.float32),
            pltpu.VMEM((N,), jnp.float32),
        ],
    )


def _tc1_body(deg_ref, x_ref, w1_ref, xs_ref, dinv_ref):
    dinv = lax.rsqrt(jnp.sum(deg_ref[...], axis=1) + 1.0)
    xw = jnp.dot(x_ref[...], w1_ref[...], preferred_element_type=jnp.float32)
    xs_ref[...] = xw * dinv[:, None]
    dinv_ref[...] = dinv[:, None]


def _tc1(deg_part, x, W1):
    return pl.pallas_call(
        _tc1_body,
        grid=(N // RB,),
        in_specs=[
            pl.BlockSpec((RB, NW), lambda j: (j, 0)),
            pl.BlockSpec((RB, D), lambda j: (j, 0)),
            pl.BlockSpec((D, H), lambda j: (0, 0)),
        ],
        out_specs=[
            pl.BlockSpec((RB, H), lambda j: (j, 0)),
            pl.BlockSpec((RB, 1), lambda j: (j, 0)),
        ],
        out_shape=[
            jax.ShapeDtypeStruct((N, H), jnp.float32),
            jax.ShapeDtypeStruct((N, 1), jnp.float32),
        ],
    )(deg_part, x, W1)


def _tc2_body(agg_ref, xs_ref, dinv_ref, b1_ref, w2_ref, zs_ref):
    dinv = dinv_ref[...]
    h = jnp.maximum(dinv * (agg_ref[...] + xs_ref[...]) + b1_ref[...], 0.0)
    z = jnp.dot(h, w2_ref[...], preferred_element_type=jnp.float32)
    zs_ref[...] = dinv * z


def _tc2(agg, xs, dinv, b1, W2):
    return pl.pallas_call(
        _tc2_body,
        grid=(N // RB,),
        in_specs=[
            pl.BlockSpec((RB, H), lambda j: (j, 0)),
            pl.BlockSpec((RB, H), lambda j: (j, 0)),
            pl.BlockSpec((RB, 1), lambda j: (j, 0)),
            pl.BlockSpec((1, H), lambda j: (0, 0)),
            pl.BlockSpec((H, 1), lambda j: (0, 0)),
        ],
        out_specs=pl.BlockSpec((RB, 1), lambda j: (j, 0)),
        out_shape=jax.ShapeDtypeStruct((N, 1), jnp.float32),
    )(agg, xs, dinv, b1, W2)


def _tc3_body(a2_ref, zs_ref, dinv_ref, b2_ref, out_ref):
    a2 = jnp.sum(a2_ref[...], axis=1)
    out_ref[...] = dinv_ref[...] * (a2[:, None] + zs_ref[...]) + b2_ref[...]


def _tc3(agg2_part, zs, dinv, b2):
    return pl.pallas_call(
        _tc3_body,
        grid=(N // RB,),
        in_specs=[
            pl.BlockSpec((RB, NW), lambda j: (j, 0)),
            pl.BlockSpec((RB, 1), lambda j: (j, 0)),
            pl.BlockSpec((RB, 1), lambda j: (j, 0)),
            pl.BlockSpec((1, 1), lambda j: (0, 0)),
        ],
        out_specs=pl.BlockSpec((RB, 1), lambda j: (j, 0)),
        out_shape=jax.ShapeDtypeStruct((N, 1), jnp.float32),
    )(agg2_part, zs, dinv, b2)


def kernel(x, edge_index, edge_weight, W1, b1, W2, b2):
    src = edge_index[0].astype(jnp.int32)
    dst = edge_index[1].astype(jnp.int32)
    ew = edge_weight.astype(jnp.float32)

    deg_part = _sc_deg()(dst, ew)
    xs, dinv = _tc1(deg_part.T, x, W1)
    xs_sc = xs.reshape(N, NW, FPT).transpose(1, 0, 2).reshape(NW, N * FPT)
    agg_sc = _sc_agg()(xs_sc, src, dst, ew)
    agg = agg_sc.reshape(NW, N, FPT).transpose(1, 0, 2).reshape(N, D)
    zs = _tc2(agg, xs, dinv, b1.reshape(1, H), W2)
    agg2_part = _sc_agg2()(zs.reshape(N), src, dst, ew)
    return _tc3(agg2_part.T, zs, dinv, b2.reshape(1, 1))
